# baseline (speedup 1.0000x reference)
import functools

import jax
import jax.numpy as jnp
from jax import lax
from jax.experimental import pallas as pl
from jax.experimental.pallas import tpu as pltpu
from jax.experimental.pallas import tpu_sc as plsc

VOCAB = 100000
EMBED_DIM = 1024
SEQ_LEN = 8192

NC = 2
NS = 16
NW = NC * NS
B_PER_W = SEQ_LEN // NW
CHUNK = 32
NCH = B_PER_W // CHUNK
NBUF = 3


def _emb_body(ids_hbm, table_hbm, out_hbm, idx_v, rows, gsems, osems):
    wid = lax.axis_index("s") * NC + lax.axis_index("c")
    base = wid * B_PER_W
    pltpu.sync_copy(ids_hbm.at[0, pl.ds(base, B_PER_W)], idx_v)
    gd = [None] * NCH
    od = [None] * NCH
    for c in range(min(NBUF, NCH)):
        gd[c] = pltpu.async_copy(
            table_hbm.at[idx_v.at[pl.ds(c * CHUNK, CHUNK)]],
            rows.at[c % NBUF], gsems[c % NBUF])
    for c in range(NCH):
        b = c % NBUF
        gd[c].wait()
        od[c] = pltpu.async_copy(rows.at[b],
                                 out_hbm.at[pl.ds(base + c * CHUNK, CHUNK)],
                                 osems[b])
        if c + NBUF < NCH:
            od[c].wait()
            gd[c + NBUF] = pltpu.async_copy(
                table_hbm.at[idx_v.at[pl.ds((c + NBUF) * CHUNK, CHUNK)]],
                rows.at[b], gsems[b])
    for c in range(max(0, NCH - NBUF), NCH):
        od[c].wait()


@jax.jit
def kernel(input_ids, embed_tokens_weight):
    call = pl.kernel(
        _emb_body,
        out_type=jax.ShapeDtypeStruct((SEQ_LEN, EMBED_DIM), jnp.float32),
        mesh=plsc.VectorSubcoreMesh(core_axis_name="c", subcore_axis_name="s"),
        scratch_types=[
            pltpu.VMEM((B_PER_W,), jnp.int32),
            pltpu.VMEM((NBUF, CHUNK, EMBED_DIM), jnp.float32),
            [pltpu.SemaphoreType.DMA] * NBUF,
            [pltpu.SemaphoreType.DMA] * NBUF,
        ],
    )
    out = call(input_ids, embed_tokens_weight)
    return out.reshape(1, SEQ_LEN, EMBED_DIM)


# speedup vs baseline: 1.0012x; 1.0012x over previous
import jax
import jax.numpy as jnp
from jax import lax
from jax.experimental import pallas as pl
from jax.experimental.pallas import tpu as pltpu
from jax.experimental.pallas import tpu_sc as plsc

VOCAB = 100000
EMBED_DIM = 1024
SEQ_LEN = 8192

NC = 2
NS = 16
NW = NC * NS
B_PER_W = SEQ_LEN // NW
CHUNK = 32
NCH = B_PER_W // CHUNK
NBUF = 3


def _emb_body(ids_hbm, table_hbm, out_hbm, idx_v, rows, gsems, osems):
    wid = lax.axis_index("s") * NC + lax.axis_index("c")
    base = wid * B_PER_W
    pltpu.sync_copy(ids_hbm.at[0, pl.ds(base, B_PER_W)], idx_v)
    gd = [None] * NCH
    od = [None] * NCH
    for c in range(min(NBUF, NCH)):
        gd[c] = pltpu.async_copy(
            table_hbm.at[idx_v.at[pl.ds(c * CHUNK, CHUNK)]],
            rows.at[c % NBUF], gsems[c % NBUF])
    for c in range(NCH):
        b = c % NBUF
        gd[c].wait()
        od[c] = pltpu.async_copy(rows.at[b],
                                 out_hbm.at[pl.ds(base + c * CHUNK, CHUNK)],
                                 osems[b])
        if c + NBUF < NCH:
            od[c].wait()
            gd[c + NBUF] = pltpu.async_copy(
                table_hbm.at[idx_v.at[pl.ds((c + NBUF) * CHUNK, CHUNK)]],
                rows.at[b], gsems[b])
    for c in range(max(0, NCH - NBUF), NCH):
        od[c].wait()


@jax.jit
def kernel(input_ids, embed_tokens_weight):
    call = pl.kernel(
        _emb_body,
        out_type=jax.ShapeDtypeStruct((SEQ_LEN, EMBED_DIM), jnp.float32),
        mesh=plsc.VectorSubcoreMesh(core_axis_name="c", subcore_axis_name="s"),
        scratch_types=[
            pltpu.VMEM((B_PER_W,), jnp.int32),
            pltpu.VMEM((NBUF, CHUNK, EMBED_DIM), jnp.float32),
            [pltpu.SemaphoreType.DMA] * NBUF,
            [pltpu.SemaphoreType.DMA] * NBUF,
        ],
    )
    out = call(input_ids, embed_tokens_weight)
    return out.reshape(1, SEQ_LEN, EMBED_DIM)
